# baseline (speedup 1.0000x reference)
import jax
import jax.numpy as jnp
from jax import lax
from jax.experimental import pallas as pl
from jax.experimental.pallas import tpu as pltpu
from jax.experimental.pallas import tpu_sc as plsc

EPS = 1e-5
B = 64
_NT = 16
_NC = 2
_K = 128
_BN = 128


def _rup(x, m):
    return (x + m - 1) // m * m


def _dot3(x, w, dn=None):
    xh = x.astype(jnp.bfloat16)
    xl = (x - xh.astype(jnp.float32)).astype(jnp.bfloat16)
    wh = w.astype(jnp.bfloat16)
    wl = (w - wh.astype(jnp.float32)).astype(jnp.bfloat16)
    if dn is None:
        dn = (((x.ndim - 1,), (0,)), ((), ()))
    f = lambda a, b: lax.dot_general(a, b, dn, preferred_element_type=jnp.float32)
    return f(xh, wh) + (f(xh, wl) + f(xl, wh))


def _sc_edge_call(Tsrc, Tattr, sidx2, tidx2, didx, h, Nd, do_counts):
    Ndp = _rup(Nd + 1, 128)
    Ep = didx.shape[0]
    Et = Ep // _NT
    nc = Et // _K
    rpt = Ndp // _NT
    nv = h // 16

    mesh = plsc.VectorSubcoreMesh(core_axis_name="c", subcore_axis_name="s")
    out_type = [jax.ShapeDtypeStruct((_NC * Ndp, h), jnp.float32),
                jax.ShapeDtypeStruct((_NC * _NT * 2 * h,), jnp.float32)]
    scratch = [
        pltpu.VMEM((_K, h), jnp.float32),
        pltpu.VMEM((_K, h), jnp.float32),
        pltpu.VMEM((_K,), jnp.int32),
        pltpu.VMEM((_K,), jnp.int32),
        pltpu.VMEM((_K,), jnp.int32),
        pltpu.VMEM((2 * h,), jnp.float32),
        pltpu.VMEM_SHARED((Ndp, h), jnp.float32),
        pltpu.SemaphoreType.DMA,
        pltpu.SemaphoreType.DMA,
    ]
    if do_counts:
        out_type.append(jax.ShapeDtypeStruct((Ndp, 16), jnp.float32))
        scratch += [pltpu.VMEM((_K, 16), jnp.float32),
                    pltpu.VMEM_SHARED((Ndp, 16), jnp.float32)]

    def body(Ts, Ta, si, ti, di, *rest):
        if do_counts:
            S_out, MSQ_out, C_out = rest[:3]
            bufA, bufB, sbuf, tbuf, dbuf, msqbuf, acc, sem1, sem2, ones, cacc = rest[3:]
        else:
            S_out, MSQ_out = rest[:2]
            bufA, bufB, sbuf, tbuf, dbuf, msqbuf, acc, sem1, sem2 = rest[2:]
        c = lax.axis_index("c")
        s = lax.axis_index("s")

        def zrow(r, carry):
            for j in range(nv):
                bufA[r, pl.ds(j * 16, 16)] = jnp.zeros((16,), jnp.float32)
            return carry
        lax.fori_loop(0, _K, zrow, 0)
        row0 = s * rpt
        off = 0
        while off < rpt:
            seg = min(_K, rpt - off)
            pltpu.sync_copy(bufA.at[pl.ds(0, seg)], acc.at[pl.ds(row0 + off, seg)])
            off += seg
        if do_counts:
            def zcrow(r, carry):
                ones[r, :] = jnp.zeros((16,), jnp.float32)
                return carry
            lax.fori_loop(0, _K, zcrow, 0)

            @pl.when(c == 0)
            def _():
                o = 0
                while o < rpt:
                    seg = min(_K, rpt - o)
                    pltpu.sync_copy(ones.at[pl.ds(0, seg)], cacc.at[pl.ds(row0 + o, seg)])
                    o += seg

            def orow(r, carry):
                ones[r, :] = jnp.ones((16,), jnp.float32)
                return carry
            lax.fori_loop(0, _K, orow, 0)
        plsc.subcore_barrier()

        ebase = c * Ep + s * Et
        dbase = s * Et

        def chunk(i, carry):
            eoff = i * _K
            pltpu.sync_copy(si.at[pl.ds(ebase + eoff, _K)], sbuf)
            pltpu.sync_copy(ti.at[pl.ds(ebase + eoff, _K)], tbuf)
            pltpu.sync_copy(di.at[pl.ds(dbase + eoff, _K)], dbuf)
            cpA = pltpu.async_copy(Ts.at[sbuf], bufA, sem1)
            cpB = pltpu.async_copy(Ta.at[tbuf], bufB, sem2)
            cpA.wait()
            cpB.wait()

            def row(k, cr):
                ms, sq = cr
                nms, nsq = [], []
                for j in range(nv):
                    a = bufA[k, pl.ds(j * 16, 16)]
                    b = bufB[k, pl.ds(j * 16, 16)]
                    m = jnp.maximum(a + b, 0.0)
                    bufA[k, pl.ds(j * 16, 16)] = m
                    nms.append(ms[j] + m)
                    nsq.append(sq[j] + m * m)
                return (tuple(nms), tuple(nsq))
            carry = lax.fori_loop(0, _K, row, carry)
            pltpu.sync_copy(bufA, acc.at[dbuf], add=True)
            if do_counts:
                @pl.when(c == 0)
                def _():
                    pltpu.sync_copy(ones, cacc.at[dbuf], add=True)
            return carry

        zero_v = tuple(jnp.zeros((16,), jnp.float32) for _ in range(nv))
        acc_ms, acc_sq = lax.fori_loop(0, nc, chunk, (zero_v, zero_v))
        plsc.subcore_barrier()

        pltpu.sync_copy(acc.at[pl.ds(row0, rpt)],
                        S_out.at[pl.ds(c * Ndp + row0, rpt)])
        for j in range(nv):
            msqbuf[pl.ds(j * 16, 16)] = acc_ms[j]
            msqbuf[pl.ds(h + j * 16, 16)] = acc_sq[j]
        wid = c * _NT + s
        pltpu.sync_copy(msqbuf, MSQ_out.at[pl.ds(wid * 2 * h, 2 * h)])
        if do_counts:
            @pl.when(c == 0)
            def _():
                pltpu.sync_copy(cacc.at[pl.ds(row0, rpt)], C_out.at[pl.ds(row0, rpt)])

    fn = pl.kernel(body, out_type=out_type, mesh=mesh, scratch_types=scratch,
                   compiler_params=pltpu.CompilerParams(use_tc_tiling_on_sc=False))
    return fn(Tsrc, Tattr, sidx2, tidx2, didx)


def _prep_edges(s, t, d, Ndp_s, Ndp_a, Ns, Na, Nd):
    E = s.shape[0]
    Ep = _rup(E, _NT * _K)
    pad = Ep - E
    s = jnp.concatenate([s.astype(jnp.int32), jnp.full((pad,), Ns, jnp.int32)])
    t = jnp.concatenate([t.astype(jnp.int32), jnp.full((pad,), Na, jnp.int32)])
    d = jnp.concatenate([d.astype(jnp.int32), jnp.full((pad,), Nd, jnp.int32)])
    sidx2 = jnp.concatenate([s, s + Ndp_s])
    tidx2 = jnp.concatenate([t, t + Ndp_a])
    return sidx2, tidx2, d


def _adj_stats(MSQ, E_real, p):
    h = MSQ.shape[0] // (_NC * _NT * 2)
    st = MSQ.reshape(_NC, _NT, 2, h).sum(axis=1)
    msum = jnp.concatenate([st[0, 0], st[1, 0]])
    sqsum = jnp.concatenate([st[0, 1], st[1, 1]])
    mean = msum / E_real
    var = sqsum / E_real - mean * mean
    alpha = p["g"] * lax.rsqrt(var + EPS)
    beta = p["bt"] - alpha * mean
    return alpha, beta


def _tables_call(x, aff_a, aff_b, Ws, bs, N_real):
    Ndp, d = x.shape
    h = d // 2
    T = len(Ws)
    TH = T * h
    nb = Ndp // _BN
    Warr = jnp.stack([jnp.concatenate([W[:, c * h:(c + 1) * h] for W in Ws], axis=1)
                      for c in range(2)])
    barr = jnp.stack([jnp.concatenate(
        [(b if b is not None else jnp.zeros((d,), jnp.float32))[c * h:(c + 1) * h]
         for b in bs]) for c in range(2)])[:, None, :]

    def bodyfn(x_ref, a_ref, b_ref, w_ref, bias_ref, *out_refs):
        i = pl.program_id(0)
        rows = lax.broadcasted_iota(jnp.int32, (_BN, 1), 0) + i * _BN
        mask = rows < N_real
        xe = a_ref[...] * x_ref[...] + b_ref[...]
        xe = jnp.where(mask, xe, 0.0)
        z = jnp.dot(xe, w_ref[0], preferred_element_type=jnp.float32)
        z = jnp.where(mask, z + bias_ref[0], 0.0)
        for j, oref in enumerate(out_refs):
            oref[...] = z[:, j * h:(j + 1) * h]

    grid = (nb, 2)
    nbb = nb
    outs = pl.pallas_call(
        bodyfn,
        grid=grid,
        in_specs=[
            pl.BlockSpec((_BN, d), lambda i, c: (i, 0)),
            pl.BlockSpec((1, d), lambda i, c: (0, 0)),
            pl.BlockSpec((1, d), lambda i, c: (0, 0)),
            pl.BlockSpec((1, d, TH), lambda i, c: (c, 0, 0)),
            pl.BlockSpec((1, 1, TH), lambda i, c: (c, 0, 0)),
        ],
        out_specs=[pl.BlockSpec((_BN, h), lambda i, c, _n=nbb: (c * _n + i, 0))
                   for _ in range(T)],
        out_shape=[jax.ShapeDtypeStruct((2 * Ndp, h), jnp.float32) for _ in range(T)],
    )(x, aff_a[None, :], aff_b[None, :], Warr, barr)
    return outs if isinstance(outs, (list, tuple)) else [outs]


def _update_call(x, aff_a, aff_b, adjs, upd, N_real):
    Ndp, d = x.shape
    h = d // 2
    nb = Ndp // _BN
    HID = upd["W1"].shape[1]

    def bodyfn(*refs):
        it = iter(refs)
        x_ref = next(it)
        a_ref = next(it)
        b_ref = next(it)
        adj_refs = [(next(it), next(it), next(it), next(it), next(it))
                    for _ in adjs]
        w1_ref, b1_ref, w2_ref, b2_ref = next(it), next(it), next(it), next(it)
        h2_ref, ps_ref, pq_ref = next(it), next(it), next(it)
        i = pl.program_id(0)
        y = a_ref[...] * x_ref[...] + b_ref[...]
        for (slo, shi, cc, al, be) in adj_refs:
            S = jnp.concatenate([slo[...], shi[...]], axis=1)
            y = y + al[...] * S + cc[:, 0:1] * be[...]
        h1 = jnp.maximum(jnp.dot(y, w1_ref[...], preferred_element_type=jnp.float32) + b1_ref[...], 0.0)
        h2 = jnp.maximum(jnp.dot(h1, w2_ref[...], preferred_element_type=jnp.float32) + b2_ref[...], 0.0)
        rows = lax.broadcasted_iota(jnp.int32, (_BN, 1), 0) + i * _BN
        maskf = (rows < N_real).astype(jnp.float32)
        h2 = h2 * maskf
        h2_ref[...] = h2
        nvalid = jnp.sum(maskf)
        bsum = jnp.sum(h2, axis=0, keepdims=True)
        bmean = bsum / nvalid
        diff = (h2 - bmean) * maskf
        ps_ref[...] = bsum[None, :, :]
        pq_ref[...] = jnp.sum(diff * diff, axis=0, keepdims=True)[None, :, :]

    in_specs = [
        pl.BlockSpec((_BN, d), lambda i: (i, 0)),
        pl.BlockSpec((1, d), lambda i: (0, 0)),
        pl.BlockSpec((1, d), lambda i: (0, 0)),
    ]
    args = [x, aff_a[None, :], aff_b[None, :]]
    for (S, C, al, be) in adjs:
        in_specs += [
            pl.BlockSpec((_BN, h), lambda i: (i, 0)),
            pl.BlockSpec((_BN, h), lambda i, _n=nb: (_n + i, 0)),
            pl.BlockSpec((_BN, 16), lambda i: (i, 0)),
            pl.BlockSpec((1, d), lambda i: (0, 0)),
            pl.BlockSpec((1, d), lambda i: (0, 0)),
        ]
        args += [S, S, C, al[None, :], be[None, :]]
    in_specs += [
        pl.BlockSpec((d, HID), lambda i: (0, 0)),
        pl.BlockSpec((1, HID), lambda i: (0, 0)),
        pl.BlockSpec((HID, HID), lambda i: (0, 0)),
        pl.BlockSpec((1, HID), lambda i: (0, 0)),
    ]
    args += [upd["W1"], upd["b1"][None, :], upd["W2"], upd["b2"][None, :]]

    h2, ps, pq = pl.pallas_call(
        bodyfn,
        grid=(nb,),
        in_specs=in_specs,
        out_specs=[pl.BlockSpec((_BN, HID), lambda i: (i, 0)),
                   pl.BlockSpec((1, 1, HID), lambda i: (i, 0, 0)),
                   pl.BlockSpec((1, 1, HID), lambda i: (i, 0, 0))],
        out_shape=[jax.ShapeDtypeStruct((Ndp, HID), jnp.float32),
                   jax.ShapeDtypeStruct((nb, 1, HID), jnp.float32),
                   jax.ShapeDtypeStruct((nb, 1, HID), jnp.float32)],
    )(*args)
    import numpy as _np
    n_b = _np.clip(N_real - _np.arange(nb) * _BN, 0, _BN).astype(_np.float32)[:, None]
    bmeans = ps[:, 0] / n_b
    mean = jnp.sum(ps[:, 0], axis=0) / N_real
    var = (jnp.sum(pq[:, 0], axis=0)
           + jnp.sum(n_b * (bmeans - mean) ** 2, axis=0)) / N_real
    a2 = upd["g"] * lax.rsqrt(var + EPS)
    b2 = upd["bt"] - a2 * mean
    return h2, a2, b2


def _pool_call(h2, aff_a, aff_b, batch_pad):
    Ndp, HID = h2.shape
    nb = Ndp // _BN

    def bodyfn(h_ref, a_ref, b_ref, bt_ref, out_ref):
        i = pl.program_id(0)

        @pl.when(i == 0)
        def _():
            out_ref[...] = jnp.zeros_like(out_ref)

        xe = a_ref[...] * h_ref[...] + b_ref[...]
        seg = lax.broadcasted_iota(jnp.int32, (_BN, B), 1)
        oh = (bt_ref[...] == seg).astype(jnp.float32)
        out_ref[...] += _dot3(oh, xe, (((0,), (0,)), ((), ())))

    return pl.pallas_call(
        bodyfn,
        grid=(nb,),
        in_specs=[pl.BlockSpec((_BN, HID), lambda i: (i, 0)),
                  pl.BlockSpec((1, HID), lambda i: (0, 0)),
                  pl.BlockSpec((1, HID), lambda i: (0, 0)),
                  pl.BlockSpec((_BN, 1), lambda i: (i, 0))],
        out_specs=pl.BlockSpec((B, HID), lambda i: (0, 0)),
        out_shape=jax.ShapeDtypeStruct((B, HID), jnp.float32),
    )(h2, aff_a[None, :], aff_b[None, :], batch_pad[:, None])


def _head(p0, p1, p2, w1, b1, w2, b2):
    def bodyfn(p0r, p1r, p2r, w1r, b1r, w2r, b2r, out_ref):
        pooled = p0r[...] + p1r[...] + p2r[...]
        hh = jnp.maximum(jnp.dot(pooled, w1r[...], preferred_element_type=jnp.float32) + b1r[...], 0.0)
        out_ref[...] = jnp.dot(hh, w2r[...], preferred_element_type=jnp.float32) + b2r[...]

    return pl.pallas_call(
        bodyfn,
        out_shape=jax.ShapeDtypeStruct((B, w2.shape[1]), jnp.float32),
    )(p0, p1, p2, w1, b1[None, :], w2, b2[None, :])


def kernel(x0, x1, x2, up_index_0, up_attr_idx_0, up_index_1, up_attr_idx_1,
           down_index_1, down_attr_idx_1, down_index_2, down_attr_idx_2,
           batch0, batch1, batch2, params):
    N0, N1, N2 = x0.shape[0], x1.shape[0], x2.shape[0]
    Np0, Np1, Np2 = _rup(N0 + 1, 128), _rup(N1 + 1, 128), _rup(N2 + 1, 128)
    E0, E1u = up_attr_idx_0.shape[0], up_attr_idx_1.shape[0]
    E1d, E2 = down_attr_idx_1.shape[0], down_attr_idx_2.shape[0]
    d0 = x0.shape[1]

    prep_u0 = _prep_edges(up_index_0[0], up_attr_idx_0, up_index_0[1], Np0, Np1, N0, N1, N0)
    prep_u1 = _prep_edges(up_index_1[0], up_attr_idx_1, up_index_1[1], Np1, Np2, N1, N2, N1)
    prep_d1 = _prep_edges(down_index_1[0], down_attr_idx_1, down_index_1[1], Np1, Np0, N1, N0, N1)
    prep_d2 = _prep_edges(down_index_2[0], down_attr_idx_2, down_index_2[1], Np2, Np1, N2, N1, N2)

    xs = [jnp.pad(x0, ((0, Np0 - N0), (0, 0))),
          jnp.pad(x1, ((0, Np1 - N1), (0, 0))),
          jnp.pad(x2, ((0, Np2 - N2), (0, 0)))]
    Ns = [N0, N1, N2]
    ident = (jnp.ones((d0,), jnp.float32), jnp.zeros((d0,), jnp.float32))
    affs = [ident, ident, ident]
    cnts = [None, None, None, None]

    for li, lp in enumerate(params["layers"]):
        d = xs[0].shape[1]
        h = d // 2
        Wu, bu = lp["up"]["W"], lp["up"]["b"]
        Wd, bd = lp["down"]["W"], lp["down"]["b"]
        t0 = _tables_call(xs[0], affs[0][0], affs[0][1],
                          [Wu[:d], Wd[d:]], [bu, None], N0)
        t1 = _tables_call(xs[1], affs[1][0], affs[1][1],
                          [Wu[d:], Wu[:d], Wd[:d], Wd[d:]],
                          [None, bu, bd, None], N1)
        t2 = _tables_call(xs[2], affs[2][0], affs[2][1],
                          [Wu[d:], Wd[:d]], [None, bd], N2)
        P_u0, Q_d1 = t0
        Q_u0, P_u1, P_d1, Q_d2 = t1
        Q_u1, P_d2 = t2

        first = li == 0
        o_u0 = _sc_edge_call(P_u0, Q_u0, *prep_u0, h=h, Nd=N0, do_counts=first)
        o_u1 = _sc_edge_call(P_u1, Q_u1, *prep_u1, h=h, Nd=N1, do_counts=first)
        o_d1 = _sc_edge_call(P_d1, Q_d1, *prep_d1, h=h, Nd=N1, do_counts=first)
        o_d2 = _sc_edge_call(P_d2, Q_d2, *prep_d2, h=h, Nd=N2, do_counts=first)
        if first:
            cnts = [o_u0[2], o_u1[2], o_d1[2], o_d2[2]]

        al0, be0 = _adj_stats(o_u0[1], E0, lp["up"])
        al1u, be1u = _adj_stats(o_u1[1], E1u, lp["up"])
        al1d, be1d = _adj_stats(o_d1[1], E1d, lp["down"])
        al2, be2 = _adj_stats(o_d2[1], E2, lp["down"])

        h20, a0, b0 = _update_call(xs[0], affs[0][0], affs[0][1],
                                   [(o_u0[0], cnts[0], al0, be0)], lp["upd"], N0)
        h21, a1, b1 = _update_call(xs[1], affs[1][0], affs[1][1],
                                   [(o_u1[0], cnts[1], al1u, be1u),
                                    (o_d1[0], cnts[2], al1d, be1d)], lp["upd"], N1)
        h22, a2, b2 = _update_call(xs[2], affs[2][0], affs[2][1],
                                   [(o_d2[0], cnts[3], al2, be2)], lp["upd"], N2)
        xs = [h20, h21, h22]
        affs = [(a0, b0), (a1, b1), (a2, b2)]

    bpad = [jnp.pad(batch0.astype(jnp.int32), (0, Np0 - N0), constant_values=-1),
            jnp.pad(batch1.astype(jnp.int32), (0, Np1 - N1), constant_values=-1),
            jnp.pad(batch2.astype(jnp.int32), (0, Np2 - N2), constant_values=-1)]
    p0 = _pool_call(xs[0], affs[0][0], affs[0][1], bpad[0])
    p1 = _pool_call(xs[1], affs[1][0], affs[1][1], bpad[1])
    p2 = _pool_call(xs[2], affs[2][0], affs[2][1], bpad[2])
    return _head(p0, p1, p2, params["lin1W"], params["lin1b"],
                 params["lin2W"], params["lin2b"])
